# Initial kernel scaffold; baseline (speedup 1.0000x reference)
import jax
import jax.numpy as jnp
from jax import lax
from jax.experimental import pallas as pl
from jax.experimental.pallas import tpu as pltpu
from jax.experimental.pallas import tpu_sc as plsc

VOCAB = 100000
EMBD = 128
B = 4096
L = 50

NC = 2
NS = 16
NW = NC * NS

TOTAL = B * L
PER_W = TOTAL // NW
CHUNK = 128
NCHUNK = PER_W // CHUNK
NBUF = 5
NGROUP = NCHUNK // NBUF


def _embed_kernel(x_hbm, table_hbm, out_hbm, idx_v, rows_v, gsem, ssem):
    wid = lax.axis_index("s") * NC + lax.axis_index("c")
    pltpu.sync_copy(x_hbm.at[pl.ds(wid * NCHUNK, NCHUNK)], idx_v)

    out_base = wid * NCHUNK

    for b in range(NBUF):
        pltpu.async_copy(table_hbm.at[idx_v.at[b]], rows_v.at[b], gsem.at[b])

    @pl.loop(0, (NGROUP - 1) * NBUF, step=NBUF)
    def _group(g0):
        for b in range(NBUF):
            g = g0 + b
            pltpu.make_async_copy(
                table_hbm.at[idx_v.at[g]], rows_v.at[b], gsem.at[b]
            ).wait()
            pltpu.async_copy(
                rows_v.at[b],
                out_hbm.at[pl.ds((out_base + g) * CHUNK, CHUNK)],
                ssem.at[b],
            )
        for b in range(NBUF):
            g = g0 + b
            pltpu.make_async_copy(
                rows_v.at[b],
                out_hbm.at[pl.ds((out_base + g) * CHUNK, CHUNK)],
                ssem.at[b],
            ).wait()
            pltpu.async_copy(
                table_hbm.at[idx_v.at[g + NBUF]], rows_v.at[b], gsem.at[b]
            )

    for b in range(NBUF):
        g = (NGROUP - 1) * NBUF + b
        pltpu.make_async_copy(
            table_hbm.at[idx_v.at[g]], rows_v.at[b], gsem.at[b]
        ).wait()
        pltpu.sync_copy(
            rows_v.at[b], out_hbm.at[pl.ds((out_base + g) * CHUNK, CHUNK)]
        )


@jax.jit
def _embed(x, table):
    x_rows = x.reshape(TOTAL // CHUNK, CHUNK)
    mesh = plsc.VectorSubcoreMesh(
        core_axis_name="c", subcore_axis_name="s", num_cores=NC,
        num_subcores=NS,
    )
    out = pl.kernel(
        _embed_kernel,
        out_type=jax.ShapeDtypeStruct((TOTAL, EMBD), jnp.float32),
        mesh=mesh,
        scratch_types=[
            pltpu.VMEM((NCHUNK, CHUNK), jnp.int32),
            pltpu.VMEM((NBUF, CHUNK, EMBD), jnp.float32),
            pltpu.SemaphoreType.DMA((NBUF,)),
            pltpu.SemaphoreType.DMA((NBUF,)),
        ],
    )(x_rows, table)
    return out.reshape(B, L, EMBD)


def kernel(x, table):
    return _embed(x.astype(jnp.int32), table)


# speedup vs baseline: 3.3065x; 3.3065x over previous
import jax
import jax.numpy as jnp
from jax import lax
from jax.experimental import pallas as pl
from jax.experimental.pallas import tpu as pltpu
from jax.experimental.pallas import tpu_sc as plsc

VOCAB = 100000
EMBD = 128
B = 4096
L = 50

NC = 2
NS = 16
NW = NC * NS

TOTAL = B * L
PER_W = TOTAL // NW
CHUNK = 128
NCHUNK = PER_W // CHUNK
NBUF = 5
NGROUP = NCHUNK // NBUF


def _embed_kernel(x_hbm, table_hbm, out_hbm, idx_v, rows_v, gsem, ssem):
    wid = lax.axis_index("s") * NC + lax.axis_index("c")
    pltpu.sync_copy(x_hbm.at[wid], idx_v)

    out_base = wid * NCHUNK

    for b in range(NBUF):
        pltpu.async_copy(table_hbm.at[idx_v.at[b]], rows_v.at[b], gsem.at[b])

    @pl.loop(0, (NGROUP - 1) * NBUF, step=NBUF)
    def _group(g0):
        for b in range(NBUF):
            g = g0 + b
            pltpu.make_async_copy(
                table_hbm.at[idx_v.at[g]], rows_v.at[b], gsem.at[b]
            ).wait()
            pltpu.async_copy(
                rows_v.at[b],
                out_hbm.at[pl.ds((out_base + g) * CHUNK, CHUNK)],
                ssem.at[b],
            )
        for b in range(NBUF):
            g = g0 + b
            pltpu.make_async_copy(
                rows_v.at[b],
                out_hbm.at[pl.ds((out_base + g) * CHUNK, CHUNK)],
                ssem.at[b],
            ).wait()
            pltpu.async_copy(
                table_hbm.at[idx_v.at[g + NBUF]], rows_v.at[b], gsem.at[b]
            )

    for b in range(NBUF):
        g = (NGROUP - 1) * NBUF + b
        pltpu.make_async_copy(
            table_hbm.at[idx_v.at[g]], rows_v.at[b], gsem.at[b]
        ).wait()
        pltpu.sync_copy(
            rows_v.at[b], out_hbm.at[pl.ds((out_base + g) * CHUNK, CHUNK)]
        )


@jax.jit
def _embed(x, table):
    x_rows = x.reshape(NW, NCHUNK, CHUNK)
    mesh = plsc.VectorSubcoreMesh(
        core_axis_name="c", subcore_axis_name="s", num_cores=NC,
        num_subcores=NS,
    )
    out = pl.kernel(
        _embed_kernel,
        out_type=jax.ShapeDtypeStruct((TOTAL, EMBD), jnp.float32),
        mesh=mesh,
        scratch_types=[
            pltpu.VMEM((NCHUNK, CHUNK), jnp.int32),
            pltpu.VMEM((NBUF, CHUNK, EMBD), jnp.float32),
            pltpu.SemaphoreType.DMA((NBUF,)),
            pltpu.SemaphoreType.DMA((NBUF,)),
        ],
    )(x_rows, table)
    return out.reshape(B, L, EMBD)


def kernel(x, table):
    return _embed(x.astype(jnp.int32), table)


# speedup vs baseline: 3.3278x; 1.0065x over previous
import jax
import jax.numpy as jnp
from jax import lax
from jax.experimental import pallas as pl
from jax.experimental.pallas import tpu as pltpu
from jax.experimental.pallas import tpu_sc as plsc

VOCAB = 100000
EMBD = 128
B = 4096
L = 50

NC = 2
NS = 16
NW = NC * NS

TOTAL = B * L
PER_W = TOTAL // NW
CHUNK = 64
NCHUNK = PER_W // CHUNK
NR = 10
LA = 5


def _embed_kernel(x_hbm, table_hbm, out_hbm, idx_v, rows_v, gsem, ssem):
    wid = lax.axis_index("s") * NC + lax.axis_index("c")
    pltpu.sync_copy(x_hbm.at[wid], idx_v)

    out_base = wid * NCHUNK

    def start_gather(g, b):
        pltpu.async_copy(table_hbm.at[idx_v.at[g]], rows_v.at[b], gsem.at[b])

    def wait_gather(g, b):
        pltpu.make_async_copy(
            table_hbm.at[idx_v.at[g]], rows_v.at[b], gsem.at[b]
        ).wait()

    def _store_desc(g, b):
        return pltpu.make_async_copy(
            rows_v.at[b],
            out_hbm.at[pl.ds((out_base + g) * CHUNK, CHUNK)],
            ssem.at[b],
        )

    def start_store(g, b):
        _store_desc(g, b).start()

    def wait_store(g, b):
        _store_desc(g, b).wait()

    for b in range(LA):
        start_gather(b, b)

    for b in range(NR):
        g = b
        wait_gather(g, b)
        start_store(g, b)
        h, hb = g + LA, (b + LA) % NR
        if g >= LA:
            wait_store(h - NR, hb)
        start_gather(h, hb)

    @pl.loop(NR, NCHUNK - NR, step=NR)
    def _pass(g0):
        for b in range(NR):
            g = g0 + b
            wait_gather(g, b)
            start_store(g, b)
            h, hb = g + LA, (b + LA) % NR
            wait_store(h - NR, hb)
            start_gather(h, hb)

    for b in range(NR):
        g = NCHUNK - NR + b
        wait_gather(g, b)
        start_store(g, b)
        h, hb = g + LA, (b + LA) % NR
        if h < NCHUNK:
            wait_store(h - NR, hb)
            start_gather(h, hb)

    for b in range(NR):
        wait_store(NCHUNK - NR + b, b)


@jax.jit
def _embed(x, table):
    x_rows = x.reshape(NW, NCHUNK, CHUNK)
    mesh = plsc.VectorSubcoreMesh(
        core_axis_name="c", subcore_axis_name="s", num_cores=NC,
        num_subcores=NS,
    )
    out = pl.kernel(
        _embed_kernel,
        out_type=jax.ShapeDtypeStruct((TOTAL, EMBD), jnp.float32),
        mesh=mesh,
        scratch_types=[
            pltpu.VMEM((NCHUNK, CHUNK), jnp.int32),
            pltpu.VMEM((NR, CHUNK, EMBD), jnp.float32),
            pltpu.SemaphoreType.DMA((NR,)),
            pltpu.SemaphoreType.DMA((NR,)),
        ],
    )(x_rows, table)
    return out.reshape(B, L, EMBD)


def kernel(x, table):
    return _embed(x.astype(jnp.int32), table)


# speedup vs baseline: 5.9477x; 1.7873x over previous
import jax
import jax.numpy as jnp
from jax import lax
from jax.experimental import pallas as pl
from jax.experimental.pallas import tpu as pltpu
from jax.experimental.pallas import tpu_sc as plsc

VOCAB = 100000
EMBD = 128
B = 4096
L = 50
LPAD = 64

NC = 2
NS = 16
NW = NC * NS

SEQ_W = B // NW
NR = 8
LA = 4


def _embed_kernel(x_hbm, table_hbm, out_hbm, idx_v, rows_v, gsem, ssem):
    wid = lax.axis_index("s") * NC + lax.axis_index("c")
    seq0 = wid * SEQ_W
    pltpu.sync_copy(x_hbm.at[pl.ds(seq0, SEQ_W)], idx_v)

    def start_gather(g, b):
        pltpu.async_copy(
            table_hbm.at[idx_v.at[g, pl.ds(0, L)]], rows_v.at[b], gsem.at[b]
        )

    def wait_gather(g, b):
        pltpu.make_async_copy(
            table_hbm.at[idx_v.at[g, pl.ds(0, L)]], rows_v.at[b], gsem.at[b]
        ).wait()

    def _store_desc(g, b):
        return pltpu.make_async_copy(
            rows_v.at[b], out_hbm.at[seq0 + g], ssem.at[b]
        )

    def start_store(g, b):
        _store_desc(g, b).start()

    def wait_store(g, b):
        _store_desc(g, b).wait()

    for b in range(LA):
        start_gather(b, b)

    for b in range(NR):
        g = b
        wait_gather(g, b)
        start_store(g, b)
        h, hb = g + LA, (b + LA) % NR
        if g >= LA:
            wait_store(h - NR, hb)
        start_gather(h, hb)

    @pl.loop(NR, SEQ_W - NR, step=NR)
    def _pass(g0):
        for b in range(NR):
            g = g0 + b
            wait_gather(g, b)
            start_store(g, b)
            h, hb = g + LA, (b + LA) % NR
            wait_store(h - NR, hb)
            start_gather(h, hb)

    for b in range(NR):
        g = SEQ_W - NR + b
        wait_gather(g, b)
        start_store(g, b)
        h, hb = g + LA, (b + LA) % NR
        if h < SEQ_W:
            wait_store(h - NR, hb)
            start_gather(h, hb)

    for b in range(NR):
        wait_store(SEQ_W - NR + b, b)


@jax.jit
def _embed(x, table):
    xp = jnp.pad(x, ((0, 0), (0, LPAD - L)))
    mesh = plsc.VectorSubcoreMesh(
        core_axis_name="c", subcore_axis_name="s", num_cores=NC,
        num_subcores=NS,
    )
    return pl.kernel(
        _embed_kernel,
        out_type=jax.ShapeDtypeStruct((B, L, EMBD), jnp.float32),
        mesh=mesh,
        scratch_types=[
            pltpu.VMEM((SEQ_W, LPAD), jnp.int32),
            pltpu.VMEM((NR, L, EMBD), jnp.float32),
            pltpu.SemaphoreType.DMA((NR,)),
            pltpu.SemaphoreType.DMA((NR,)),
        ],
    )(xp, table)


def kernel(x, table):
    return _embed(x.astype(jnp.int32), table)
